# baseline (speedup 1.0000x reference)
import jax
import jax.numpy as jnp
from jax.experimental import pallas as pl

_BM = 4096


def _split_cat(x):
    hi = x.astype(jnp.bfloat16)
    lo = (x - hi.astype(jnp.float32)).astype(jnp.bfloat16)
    return jnp.concatenate([hi, lo], axis=1)


def _layer(x, w_ref, b_ref):
    n = b_ref.shape[1]
    r = jnp.dot(_split_cat(x), w_ref[...], preferred_element_type=jnp.float32)
    return jnp.tanh(r[:, :n] + r[:, n:] + b_ref[...])


def _ffn_block(obs_ref, w1_ref, w2_ref, b1_ref, b2_ref, out_ref):
    y = _layer(_layer(obs_ref[...], w1_ref, b1_ref), w2_ref, b2_ref)
    pack = 128 // y.shape[1]
    y3 = y.reshape(y.shape[0] // pack, pack, y.shape[1])
    out_ref[...] = jnp.concatenate([y3[:, s, :] for s in range(pack)], axis=1)


def _pack_weights(w):
    hi = w.astype(jnp.bfloat16)
    lo = (w - hi.astype(jnp.float32)).astype(jnp.bfloat16)
    half = jnp.concatenate([hi, lo], axis=1)
    return jnp.concatenate([half, half], axis=0)


def kernel(obs, W1, W2, b1, b2):
    if obs.ndim == 1:
        obs = obs[None, :]
    batch, n_in = obs.shape
    n_hid = W1.shape[1]
    n_out = W2.shape[1]
    w1p = _pack_weights(W1)
    w2p = _pack_weights(W2)
    bm = min(_BM, batch)
    grid = (pl.cdiv(batch, bm),)
    rep = lambda i: (0, 0)
    pack = 128 // n_out
    out = pl.pallas_call(
        _ffn_block,
        grid=grid,
        in_specs=[
            pl.BlockSpec((bm, n_in), lambda i: (i, 0)),
            pl.BlockSpec((2 * n_in, 2 * n_hid), rep),
            pl.BlockSpec((2 * n_hid, 2 * n_out), rep),
            pl.BlockSpec((1, n_hid), rep),
            pl.BlockSpec((1, n_out), rep),
        ],
        out_specs=pl.BlockSpec((bm // pack, 128), lambda i: (i, 0)),
        out_shape=jax.ShapeDtypeStruct((batch // pack, 128), jnp.float32),
    )(obs, w1p, w2p, b1[None, :], b2[None, :])
    return out.reshape(batch, n_out)


# speedup vs baseline: 1.2085x; 1.2085x over previous
import jax
import jax.numpy as jnp
from jax.experimental import pallas as pl
from jax.experimental.pallas import tpu as pltpu

_BM = 2048


def _split_cat(x):
    hi = x.astype(jnp.bfloat16)
    lo = (x - hi.astype(jnp.float32)).astype(jnp.bfloat16)
    return jnp.concatenate([hi, lo], axis=1)


def _ffn_block(obs_ref, w1_ref, w2_ref, b1_ref, b2_ref, out_ref):
    n_hid = b1_ref.shape[1]
    n_out = b2_ref.shape[1]
    a1 = _split_cat(obs_ref[...])
    r1 = jnp.dot(a1, w1_ref[...], preferred_element_type=jnp.float32)
    h = jnp.tanh(r1[:, :n_hid] + r1[:, n_hid:] + b1_ref[...])
    a2 = _split_cat(h)
    r2 = jnp.dot(a2, w2_ref[...], preferred_element_type=jnp.float32)
    out_ref[...] = jnp.tanh(r2[:, :n_out] + r2[:, n_out:] + b2_ref[...])


def _pack_weights(w):
    hi = w.astype(jnp.bfloat16)
    lo = (w - hi.astype(jnp.float32)).astype(jnp.bfloat16)
    half = jnp.concatenate([hi, lo], axis=1)
    return jnp.concatenate([half, half], axis=0)


def kernel(obs, W1, W2, b1, b2):
    if obs.ndim == 1:
        obs = obs[None, :]
    batch, n_in = obs.shape
    n_hid = W1.shape[1]
    n_out = W2.shape[1]
    w1p = _pack_weights(W1)
    w2p = _pack_weights(W2)
    bm = min(_BM, batch)
    grid = (pl.cdiv(batch, bm),)
    rep = lambda i: (0, 0)
    return pl.pallas_call(
        _ffn_block,
        grid=grid,
        in_specs=[
            pl.BlockSpec((bm, n_in), lambda i: (i, 0)),
            pl.BlockSpec((2 * n_in, 2 * n_hid), rep),
            pl.BlockSpec((2 * n_hid, 2 * n_out), rep),
            pl.BlockSpec((1, n_hid), rep),
            pl.BlockSpec((1, n_out), rep),
        ],
        out_specs=pl.BlockSpec((bm, n_out), lambda i: (i, 0)),
        out_shape=jax.ShapeDtypeStruct((batch, n_out), jnp.float32),
        compiler_params=pltpu.CompilerParams(dimension_semantics=(pltpu.PARALLEL,)),
    )(obs, w1p, w2p, b1[None, :], b2[None, :])


# speedup vs baseline: 1.8227x; 1.5082x over previous
import jax
import jax.numpy as jnp
from jax.experimental import pallas as pl

_BM = 2048


def _split_cat0(x):
    hi = x.astype(jnp.bfloat16)
    lo = (x - hi.astype(jnp.float32)).astype(jnp.bfloat16)
    return jnp.concatenate([hi, lo], axis=0)


def _layer_t(xT, wt_ref, b_ref):
    n = wt_ref.shape[0] // 2
    r = jnp.dot(wt_ref[...], _split_cat0(xT), preferred_element_type=jnp.float32)
    return jnp.tanh(r[:n] + r[n:] + b_ref[...])


def _ffn_block(obs_ref, w1t_ref, w2t_ref, b1_ref, b2_ref, out_ref):
    obsT = obs_ref[...].T
    hT = _layer_t(obsT, w1t_ref, b1_ref)
    out_ref[...] = _layer_t(hT, w2t_ref, b2_ref)


def _pack_weights_t(w):
    hi = w.astype(jnp.bfloat16)
    lo = (w - hi.astype(jnp.float32)).astype(jnp.bfloat16)
    half = jnp.concatenate([hi, lo], axis=1)
    packed = jnp.concatenate([half, half], axis=0)
    return packed.T


def kernel(obs, W1, W2, b1, b2):
    if obs.ndim == 1:
        obs = obs[None, :]
    batch, n_in = obs.shape
    n_hid = W1.shape[1]
    n_out = W2.shape[1]
    w1t = _pack_weights_t(W1)
    w2t = _pack_weights_t(W2)
    bm = min(_BM, batch)
    grid = (pl.cdiv(batch, bm),)
    rep = lambda i: (0, 0)
    out = pl.pallas_call(
        _ffn_block,
        grid=grid,
        in_specs=[
            pl.BlockSpec((bm, n_in), lambda i: (i, 0)),
            pl.BlockSpec((2 * n_hid, 2 * n_in), rep),
            pl.BlockSpec((2 * n_out, 2 * n_hid), rep),
            pl.BlockSpec((n_hid, 1), rep),
            pl.BlockSpec((n_out, 1), rep),
        ],
        out_specs=pl.BlockSpec((n_out, bm), lambda i: (0, i)),
        out_shape=jax.ShapeDtypeStruct((n_out, batch), jnp.float32),
    )(obs, w1t, w2t, b1[:, None], b2[:, None])
    return out.T


# speedup vs baseline: 2.1151x; 1.1604x over previous
import jax
import jax.numpy as jnp
from jax.experimental import pallas as pl

_BM = 4096


def _split_cat0(x):
    hi = x.astype(jnp.bfloat16)
    lo = (x - hi.astype(jnp.float32)).astype(jnp.bfloat16)
    return jnp.concatenate([hi, lo], axis=0)


def _layer_t(xT, wt_ref, b_ref):
    n = wt_ref.shape[0] // 2
    r = jnp.dot(wt_ref[...], _split_cat0(xT), preferred_element_type=jnp.float32)
    return jnp.tanh(r[:n] + r[n:] + b_ref[...])


def _ffn_block(obs_ref, w1t_ref, w2t_ref, b1_ref, b2_ref, out_ref):
    obsT = obs_ref[...].T
    hT = _layer_t(obsT, w1t_ref, b1_ref)
    out_ref[...] = _layer_t(hT, w2t_ref, b2_ref)


def _pack_weights_t(w):
    hi = w.astype(jnp.bfloat16)
    lo = (w - hi.astype(jnp.float32)).astype(jnp.bfloat16)
    half = jnp.concatenate([hi, lo], axis=1)
    packed = jnp.concatenate([half, half], axis=0)
    return packed.T


def kernel(obs, W1, W2, b1, b2):
    if obs.ndim == 1:
        obs = obs[None, :]
    batch, n_in = obs.shape
    n_hid = W1.shape[1]
    n_out = W2.shape[1]
    w1t = _pack_weights_t(W1)
    w2t = _pack_weights_t(W2)
    bm = min(_BM, batch)
    grid = (pl.cdiv(batch, bm),)
    rep = lambda i: (0, 0)
    out = pl.pallas_call(
        _ffn_block,
        grid=grid,
        in_specs=[
            pl.BlockSpec((bm, n_in), lambda i: (i, 0)),
            pl.BlockSpec((2 * n_hid, 2 * n_in), rep),
            pl.BlockSpec((2 * n_out, 2 * n_hid), rep),
            pl.BlockSpec((n_hid, 1), rep),
            pl.BlockSpec((n_out, 1), rep),
        ],
        out_specs=pl.BlockSpec((n_out, bm), lambda i: (0, i)),
        out_shape=jax.ShapeDtypeStruct((n_out, batch), jnp.float32),
    )(obs, w1t, w2t, b1[:, None], b2[:, None])
    return out.T


# speedup vs baseline: 2.1525x; 1.0177x over previous
import jax
import jax.numpy as jnp
from jax.experimental import pallas as pl

_BM = 8192


def _split_cat0(x):
    hi = x.astype(jnp.bfloat16)
    lo = (x - hi.astype(jnp.float32)).astype(jnp.bfloat16)
    return jnp.concatenate([hi, lo], axis=0)


def _layer_t(xT, wt_ref, b_ref):
    n = wt_ref.shape[0] // 2
    r = jnp.dot(wt_ref[...], _split_cat0(xT), preferred_element_type=jnp.float32)
    return jnp.tanh(r[:n] + r[n:] + b_ref[...])


def _ffn_block(obs_ref, w1t_ref, w2t_ref, b1_ref, b2_ref, out_ref):
    obsT = obs_ref[...].T
    hT = _layer_t(obsT, w1t_ref, b1_ref)
    out_ref[...] = _layer_t(hT, w2t_ref, b2_ref)


def _pack_weights_t(w):
    hi = w.astype(jnp.bfloat16)
    lo = (w - hi.astype(jnp.float32)).astype(jnp.bfloat16)
    half = jnp.concatenate([hi, lo], axis=1)
    packed = jnp.concatenate([half, half], axis=0)
    return packed.T


def kernel(obs, W1, W2, b1, b2):
    if obs.ndim == 1:
        obs = obs[None, :]
    batch, n_in = obs.shape
    n_hid = W1.shape[1]
    n_out = W2.shape[1]
    w1t = _pack_weights_t(W1)
    w2t = _pack_weights_t(W2)
    bm = min(_BM, batch)
    grid = (pl.cdiv(batch, bm),)
    rep = lambda i: (0, 0)
    out = pl.pallas_call(
        _ffn_block,
        grid=grid,
        in_specs=[
            pl.BlockSpec((bm, n_in), lambda i: (i, 0)),
            pl.BlockSpec((2 * n_hid, 2 * n_in), rep),
            pl.BlockSpec((2 * n_out, 2 * n_hid), rep),
            pl.BlockSpec((n_hid, 1), rep),
            pl.BlockSpec((n_out, 1), rep),
        ],
        out_specs=pl.BlockSpec((n_out, bm), lambda i: (0, i)),
        out_shape=jax.ShapeDtypeStruct((n_out, batch), jnp.float32),
    )(obs, w1t, w2t, b1[:, None], b2[:, None])
    return out.T
